# baseline (speedup 1.0000x reference)
import jax
import jax.numpy as jnp
from jax import lax
from jax.experimental import pallas as pl
from jax.experimental.pallas import tpu as pltpu
from jax.experimental.pallas import tpu_sc as plsc

_NUM_EMB = 100000
_DIM = 64
_N_SPECIAL = 16
_NF4 = [-1.0, -0.6962, -0.5251, -0.3949, -0.2844, -0.1848, -0.0911, 0.0,
        0.0796, 0.1609, 0.2461, 0.3379, 0.4407, 0.5626, 0.723, 1.0]

_NC = 2
_NS = 16
_NW = _NC * _NS
_L = 16

_NSEQ = 4096
_SEQ = 50
_TOKENS = _NSEQ * _SEQ
_PER_W = _TOKENS // _NW
_CHUNK = 200
_NCHUNK = _PER_W // _CHUNK
_SEQ_PER_CHUNK = _CHUNK // _SEQ
_SEQ_PER_W = _PER_W // _SEQ
_GROUP_STARTS = list(range(0, _CHUNK - _L + 1, _L))
if _CHUNK % _L:
    _GROUP_STARTS.append(_CHUNK - _L)
_FLAT_STARTS = (0, 16, 32, 34)


def _body(spec_hbm, qp_hbm, ids_hbm, lev_hbm, out_hbm,
          ids2_v, ids_v, pix0, pix1, q0, q1, o_b, lev_v, spec_v,
          isem0, isem1, osem):
    wid = lax.axis_index("s") * _NC + lax.axis_index("c")
    pltpu.sync_copy(lev_hbm, lev_v)
    pltpu.sync_copy(spec_hbm, spec_v)
    pltpu.sync_copy(ids_hbm.at[pl.ds(wid * _SEQ_PER_W, _SEQ_PER_W)], ids2_v)
    for s in range(_SEQ_PER_W):
        for c0 in _FLAT_STARTS:
            ids_v[pl.ds(s * _SEQ + c0, _L)] = ids2_v[s, pl.ds(c0, _L)]

    lane = lax.iota(jnp.int32, _L)
    levels = lev_v[...]
    dnums = lax.GatherDimensionNumbers(
        offset_dims=(), collapsed_slice_dims=(0,), start_index_map=(0,))

    def lut(q):
        return lax.gather(levels, q[:, None], dnums, (1,),
                          mode=lax.GatherScatterMode.PROMISE_IN_BOUNDS)

    bufs = ((pix0, q0, isem0), (pix1, q1, isem1))

    def issue(i, buf):
        pix_b, q_b, isem = buf
        off = i * _CHUNK
        for r0 in _GROUP_STARTS:
            pix_b[pl.ds(r0, _L)] = lax.shift_right_logical(
                ids_v[pl.ds(off + r0, _L)], 2)
        pltpu.async_copy(qp_hbm.at[pix_b], q_b, isem)

    def wait_in(buf):
        pix_b, q_b, isem = buf
        pltpu.make_async_copy(qp_hbm.at[pix_b], q_b, isem).wait()

    def wait_out():
        for s in range(_SEQ_PER_CHUNK):
            pltpu.make_async_copy(
                o_b.at[pl.ds(s * _SEQ, _SEQ)],
                out_hbm.at[wid * _SEQ_PER_W + s], osem).wait()

    def compute(i, buf):
        pix_b, q_b, _ = buf
        off = i * _CHUNK

        def dequant_group(row0):
            ids_vec = ids_v[pl.ds(off + row0, _L)]
            for t in range(_L):
                row = row0 + t
                qtr = (ids_vec[t] & 3) * 32
                qw = q_b[row, pl.ds(qtr, _L)]
                sw = q_b[row, pl.ds(qtr + _L, _L)]
                scv = jnp.full((_L,), plsc.bitcast(sw, jnp.float32)[0])
                for bb in range(4):
                    codes = lax.shift_right_logical(qw, 8 * bb) & 0xFF
                    o_b[row, pl.ds(bb * _L, _L)] = lut(codes) * scv

        @plsc.parallel_loop(0, _CHUNK // _L)
        def group_body(g):
            dequant_group(g * _L)

        if _CHUNK % _L:
            dequant_group(_CHUNK - _L)

        def patch_group(row0):
            ids_vec = ids_v[pl.ds(off + row0, _L)]
            nsp = jnp.sum(jnp.where(ids_vec < _N_SPECIAL, 1, 0))

            @pl.when(nsp > 0)
            def _patch():
                for t in range(_L):
                    tid = ids_vec[t]

                    @pl.when(tid < _N_SPECIAL)
                    def _one():
                        tsplat = jnp.full((_L,), tid, jnp.int32)
                        rsplat = jnp.full((_L,), row0 + t, jnp.int32)
                        for cc in range(_DIM // _L):
                            col = cc * _L + lane
                            v = plsc.load_gather(spec_v, [tsplat, col])
                            plsc.store_scatter(o_b, [rsplat, col], v)

        def patch_body(g, carry2):
            patch_group(g * _L)
            return carry2

        lax.fori_loop(0, _CHUNK // _L, patch_body, 0)
        if _CHUNK % _L:
            patch_group(_CHUNK - _L)

    def store_out(i):
        seq0 = wid * _SEQ_PER_W + i * _SEQ_PER_CHUNK
        for s in range(_SEQ_PER_CHUNK):
            pltpu.async_copy(o_b.at[pl.ds(s * _SEQ, _SEQ)],
                             out_hbm.at[seq0 + s], osem)

    issue(0, bufs[0])

    def pair_body(kk, carry):
        for b in (0, 1):
            i = kk * 2 + b
            buf = bufs[b]

            @pl.when(i + 1 < _NCHUNK)
            def _prefetch():
                issue(i + 1, bufs[1 - b])

            wait_in(buf)

            @pl.when(i >= 1)
            def _drain():
                wait_out()

            compute(i, buf)
            store_out(i)
        return carry

    lax.fori_loop(0, _NCHUNK // 2, pair_body, 0)
    wait_out()


@jax.jit
def _run(special_embeddings, q_packed, ids2d, levels):
    mesh = plsc.VectorSubcoreMesh(core_axis_name="c", subcore_axis_name="s",
                                  num_cores=_NC, num_subcores=_NS)
    fn = pl.kernel(
        _body,
        out_type=jax.ShapeDtypeStruct((_NSEQ, _SEQ, _DIM), jnp.float32),
        mesh=mesh,
        scratch_types=[
            pltpu.VMEM((_SEQ_PER_W, _SEQ), jnp.int32),
            pltpu.VMEM((_PER_W,), jnp.int32),
            pltpu.VMEM((_CHUNK,), jnp.int32),
            pltpu.VMEM((_CHUNK,), jnp.int32),
            pltpu.VMEM((_CHUNK, 128), jnp.int32),
            pltpu.VMEM((_CHUNK, 128), jnp.int32),
            pltpu.VMEM((_CHUNK, _DIM), jnp.float32),
            pltpu.VMEM((_L,), jnp.float32),
            pltpu.VMEM((_N_SPECIAL, 2 * _DIM), jnp.float32),
            pltpu.SemaphoreType.DMA,
            pltpu.SemaphoreType.DMA,
            pltpu.SemaphoreType.DMA,
        ],
        compiler_params=pltpu.CompilerParams(needs_layout_passes=False,
                                             use_tc_tiling_on_sc=True),
    )
    return fn(special_embeddings, q_packed, ids2d, levels)


_COL_PERM = tuple((j // 4) + 16 * (j % 4) for j in range(_DIM))


def kernel(main_scales, special_embeddings, main_quantized, special_indices,
           input_ids):
    del special_indices
    ids2d = input_ids.astype(jnp.int32)
    codes = main_quantized[:, jnp.asarray(_COL_PERM)].astype(jnp.uint8)
    words = lax.bitcast_convert_type(
        codes.reshape(_NUM_EMB, _L, 4), jnp.int32)
    scale_w = lax.bitcast_convert_type(main_scales, jnp.int32)[:, None]
    row32 = jnp.concatenate(
        [words, scale_w, jnp.zeros((_NUM_EMB, 15), jnp.int32)], axis=1)
    q_packed = row32.reshape(_NUM_EMB // 4, 128)
    levels = jnp.asarray(_NF4, dtype=jnp.float32)
    spec_pad = jnp.pad(special_embeddings.astype(jnp.float32),
                       ((0, 0), (0, _DIM)))
    return _run(spec_pad, q_packed, ids2d, levels)


# speedup vs baseline: 1.5541x; 1.5541x over previous
import functools

import jax
import jax.numpy as jnp
from jax import lax
from jax.experimental import pallas as pl
from jax.experimental.pallas import tpu as pltpu
from jax.experimental.pallas import tpu_sc as plsc

_NUM_EMB = 100000
_DIM = 64
_N_SPECIAL = 16
_NF4 = [-1.0, -0.6962, -0.5251, -0.3949, -0.2844, -0.1848, -0.0911, 0.0,
        0.0796, 0.1609, 0.2461, 0.3379, 0.4407, 0.5626, 0.723, 1.0]

_NC = 2
_NS = 16
_NW = _NC * _NS
_L = 16

_TOKENS = 204800
_PER_W = _TOKENS // _NW
_CHUNK = 400
_NCHUNK = _PER_W // _CHUNK
_SEQ_PER_CHUNK = _CHUNK // 50
_SEQ_PER_W = _PER_W // 50


def _body(scales_hbm, spec_hbm, q_hbm, ids_hbm, lev_hbm, out_hbm,
          ids0, ids1, q0, q1, s0, s1, o0, o1, lev_v, spec_v,
          isem0, isem1, osem0, osem1):
    wid = lax.axis_index("s") * _NC + lax.axis_index("c")
    base = wid * _PER_W
    pltpu.sync_copy(lev_hbm, lev_v)
    pltpu.sync_copy(spec_hbm, spec_v)

    lane = lax.iota(jnp.int32, _L)
    levels = lev_v[...]
    dnums = lax.GatherDimensionNumbers(
        offset_dims=(), collapsed_slice_dims=(0,), start_index_map=(0,))

    def lut(q):
        return lax.gather(levels, q[:, None], dnums, (1,),
                          mode=lax.GatherScatterMode.PROMISE_IN_BOUNDS)

    bufs = ((ids0, q0, s0, o0, isem0, osem0),
            (ids1, q1, s1, o1, isem1, osem1))

    def issue(i, buf):
        ids_b, q_b, s_b, _, isem, _ = buf
        off = base + i * _CHUNK
        pltpu.sync_copy(ids_hbm.at[pl.ds(off, _CHUNK)], ids_b)
        pltpu.async_copy(q_hbm.at[ids_b], q_b, isem)
        pltpu.async_copy(scales_hbm.at[ids_b], s_b, isem)

    def wait_in(buf):
        ids_b, q_b, s_b, _, isem, _ = buf
        pltpu.make_async_copy(q_hbm.at[ids_b], q_b, isem).wait()
        pltpu.make_async_copy(scales_hbm.at[ids_b], s_b, isem).wait()

    def wait_out(buf):
        o_b, osem = buf[3], buf[5]
        for s in range(_SEQ_PER_CHUNK):
            pltpu.make_async_copy(o_b.at[pl.ds(s * 50, 50)],
                                  out_hbm.at[wid * _SEQ_PER_W + s],
                                  osem).wait()

    def compute(buf):
        ids_b, q_b, s_b, o_b = buf[0], buf[1], buf[2], buf[3]

        @plsc.parallel_loop(0, _CHUNK // _L)
        def group_body(g):
            row0 = g * _L
            svec = s_b[pl.ds(row0, _L)]
            for t in range(_L):
                row = row0 + t
                scv = jnp.full((_L,), svec[t])
                for c4 in range(_DIM // _L):
                    q = q_b[row, pl.ds(c4 * _L, _L)]
                    o_b[row, pl.ds(c4 * _L, _L)] = lut(q) * scv

        def patch_body(g, carry2):
            row0 = g * _L
            ids_vec = ids_b[pl.ds(row0, _L)]
            nsp = jnp.sum(jnp.where(ids_vec < _N_SPECIAL, 1, 0))

            @pl.when(nsp > 0)
            def _patch():
                for t in range(_L):
                    tid = ids_vec[t]

                    @pl.when(tid < _N_SPECIAL)
                    def _one():
                        tsplat = jnp.full((_L,), tid, jnp.int32)
                        rsplat = jnp.full((_L,), row0 + t, jnp.int32)
                        for cc in range(_DIM // _L):
                            col = cc * _L + lane
                            v = plsc.load_gather(spec_v, [tsplat, col])
                            plsc.store_scatter(o_b, [rsplat, col], v)

            return carry2

        lax.fori_loop(0, _CHUNK // _L, patch_body, 0)

    def store_out(i, buf):
        o_b, osem = buf[3], buf[5]
        seq0 = wid * _SEQ_PER_W + i * _SEQ_PER_CHUNK
        for s in range(_SEQ_PER_CHUNK):
            pltpu.async_copy(o_b.at[pl.ds(s * 50, 50)],
                             out_hbm.at[seq0 + s], osem)

    issue(0, bufs[0])

    def pair_body(kk, carry):
        for b in (0, 1):
            i = kk * 2 + b
            buf = bufs[b]

            @pl.when(i + 1 < _NCHUNK)
            def _prefetch():
                issue(i + 1, bufs[1 - b])

            wait_in(buf)

            @pl.when(i >= 2)
            def _drain():
                wait_out(buf)

            compute(buf)
            store_out(i, buf)
        return carry

    lax.fori_loop(0, _NCHUNK // 2, pair_body, 0)
    wait_out(bufs[0])
    wait_out(bufs[1])


@jax.jit
def _run(main_scales, special_embeddings, main_quantized, ids_flat, levels):
    mesh = plsc.VectorSubcoreMesh(core_axis_name="c", subcore_axis_name="s",
                                  num_cores=_NC, num_subcores=_NS)
    fn = pl.kernel(
        _body,
        out_type=jax.ShapeDtypeStruct((_TOKENS // 50, 50, _DIM),
                                      jnp.float32),
        mesh=mesh,
        scratch_types=[
            pltpu.VMEM((_CHUNK,), jnp.int32),
            pltpu.VMEM((_CHUNK,), jnp.int32),
            pltpu.VMEM((_CHUNK, _DIM), jnp.int32),
            pltpu.VMEM((_CHUNK, _DIM), jnp.int32),
            pltpu.VMEM((_CHUNK,), jnp.float32),
            pltpu.VMEM((_CHUNK,), jnp.float32),
            pltpu.VMEM((_CHUNK, _DIM), jnp.float32),
            pltpu.VMEM((_CHUNK, _DIM), jnp.float32),
            pltpu.VMEM((_L,), jnp.float32),
            pltpu.VMEM((_N_SPECIAL, _DIM), jnp.float32),
            pltpu.SemaphoreType.DMA,
            pltpu.SemaphoreType.DMA,
            pltpu.SemaphoreType.DMA,
            pltpu.SemaphoreType.DMA,
        ],
        compiler_params=pltpu.CompilerParams(needs_layout_passes=False,
                                             use_tc_tiling_on_sc=False),
    )
    return fn(main_scales, special_embeddings, main_quantized, ids_flat,
              levels)


def kernel(main_scales, special_embeddings, main_quantized, special_indices,
           input_ids):
    del special_indices
    ids_flat = input_ids.reshape(-1).astype(jnp.int32)
    levels = jnp.asarray(_NF4, dtype=jnp.float32)
    return _run(main_scales, special_embeddings.astype(jnp.float32),
                main_quantized, ids_flat, levels)
